# Initial kernel scaffold; baseline (speedup 1.0000x reference)
import jax
import jax.numpy as jnp
from jax.experimental import pallas as pl


def kernel(x, adj, c, W1, b1, cheb_w, cheb_b):
    raise NotImplementedError("write your pallas kernel here")


# speedup vs baseline: 3.8483x; 3.8483x over previous
import jax
import jax.numpy as jnp
from jax.experimental import pallas as pl
from jax.experimental.pallas import tpu as pltpu


def _body(x_ref, a_ref, c_ref, w1_ref, b1_ref, wbig_ref, bias_ref, o_ref, xall):
    B, N, D = x_ref.shape
    for b in range(B):
        cb = c_ref[b:b + 1, :]
        sc = cb * jax.nn.sigmoid(cb)
        mod = jnp.dot(sc, w1_ref[:, :], preferred_element_type=jnp.float32)
        mod = mod + b1_ref[0:1, :]
        shift = mod[:, :D]
        scale = mod[:, D:]
        xb = x_ref[b]
        mu = jnp.mean(xb, axis=1, keepdims=True)
        xc = xb - mu
        var = jnp.mean(xc * xc, axis=1, keepdims=True)
        xn = xc * jax.lax.rsqrt(var + 1e-6)
        xall[:, D * b:D * (b + 1)] = xn * (1.0 + scale) + shift

    zall = jnp.dot(xall[:, :], wbig_ref[:, :], preferred_element_type=jnp.float32)
    z0 = zall[:, 0:128]
    z1 = zall[:, 128:256]
    z2 = zall[:, 256:384]

    a = a_ref[:, :]
    d = jnp.sum(a, axis=1, keepdims=True)
    s = jax.lax.rsqrt(d)

    def lap(y):
        return y - s * jnp.dot(a, s * y, preferred_element_type=jnp.float32)

    t = lap(z2)
    w = lap(z1 + 2.0 * t)
    o_ref[:, :] = z0 - z2 + w + bias_ref[0:1, :]


def kernel(x, adj, c, W1, b1, cheb_w, cheb_b):
    B, N, D = x.shape
    K, _, _, OUT = cheb_w.shape
    PAD = 16

    c2 = c.reshape(B, D)
    b1r = b1.reshape(1, 2 * D)
    w16 = jnp.pad(cheb_w[:, 0], ((0, 0), (0, 0), (0, PAD - OUT)))
    eye = jnp.eye(B, dtype=x.dtype)
    wbig = jnp.concatenate([jnp.kron(eye, w16[k]) for k in range(K)], axis=1)
    bias128 = jnp.tile(jnp.pad(cheb_b.reshape(OUT), (0, PAD - OUT)), B)
    bias128 = bias128.reshape(1, B * PAD)

    out_full = pl.pallas_call(
        _body,
        out_shape=jax.ShapeDtypeStruct((N, B * PAD), jnp.float32),
        scratch_shapes=[pltpu.VMEM((N, B * D), jnp.float32)],
        compiler_params=pltpu.CompilerParams(
            vmem_limit_bytes=100 * 1024 * 1024,
        ),
    )(x, adj, c2, W1, b1r, wbig, bias128)

    out = out_full.reshape(N, B, PAD)[:, :, :OUT].transpose(1, 0, 2)
    return out
